# baseline (speedup 1.0000x reference)
import functools

import jax
import jax.numpy as jnp
from jax import lax
from jax.experimental import pallas as pl
from jax.experimental.pallas import tpu as pltpu
from jax.experimental.pallas import tpu_sc as plsc

VOCAB = 30000
EMB = 128
B = 4
S = 2048
EPS = 1e-12

NC = 2
NS = 16
NW = NC * NS
TOK = B * S
TPW = TOK // NW
IDXW = 128
NIDX = TPW // IDXW

TC_BLK = 512


@functools.partial(
    pl.kernel,
    out_type=jax.ShapeDtypeStruct((TOK, EMB), jnp.float32),
    mesh=plsc.VectorSubcoreMesh(core_axis_name="c", subcore_axis_name="s"),
    scratch_types=[
        pltpu.VMEM((NIDX, IDXW), jnp.int32),
        pltpu.VMEM((TPW, EMB), jnp.float32),
        pltpu.SemaphoreType.DMA,
    ],
)
def _gather_add(ids_hbm, w_hbm, pos_hbm, out_hbm, idx_v, buf_v, gsem):
    cid = lax.axis_index("c")
    sid = lax.axis_index("s")
    wid = sid * NC + cid
    base = wid * TPW
    pos_base = lax.rem(base, S)

    pltpu.sync_copy(ids_hbm.at[pl.ds(wid * NIDX, NIDX)], idx_v)
    pltpu.sync_copy(pos_hbm.at[pl.ds(pos_base, TPW)], buf_v)

    cps = [
        pltpu.async_copy(w_hbm.at[idx_v.at[j]],
                         buf_v.at[pl.ds(j * IDXW, IDXW)], gsem, add=True)
        for j in range(NIDX)
    ]
    for cp in cps:
        cp.wait()

    pltpu.sync_copy(buf_v, out_hbm.at[pl.ds(base, TPW)])


def _ln_body(mid_ref, tte_ref, g_ref, b_ref, o_ref):
    x = mid_ref[...] + tte_ref[...]
    m = jnp.mean(x, axis=-1, keepdims=True)
    d = x - m
    var = jnp.mean(d * d, axis=-1, keepdims=True)
    o_ref[...] = d * lax.rsqrt(var + EPS) * g_ref[...] + b_ref[...]


_ln_call = pl.pallas_call(
    _ln_body,
    out_shape=jax.ShapeDtypeStruct((TOK, EMB), jnp.float32),
    grid=(TOK // TC_BLK,),
    in_specs=[
        pl.BlockSpec((TC_BLK, EMB), lambda i: (i, 0)),
        pl.BlockSpec((1, EMB), lambda i: (0, 0)),
        pl.BlockSpec((1, EMB), lambda i: (0, 0)),
        pl.BlockSpec((1, EMB), lambda i: (0, 0)),
    ],
    out_specs=pl.BlockSpec((TC_BLK, EMB), lambda i: (i, 0)),
)


def kernel(input_ids, weight, token_type_embeddings, position_embeddings,
           ln_gamma, ln_beta):
    ids = input_ids.astype(jnp.int32).reshape(TOK // IDXW, IDXW)
    mid = _gather_add(ids, weight, position_embeddings)
    out = _ln_call(mid,
                   token_type_embeddings[0].reshape(1, EMB),
                   ln_gamma.reshape(1, EMB),
                   ln_beta.reshape(1, EMB))
    return out.reshape(B, S, EMB)


# speedup vs baseline: 1.1897x; 1.1897x over previous
import functools

import jax
import jax.numpy as jnp
from jax import lax
from jax.experimental import pallas as pl
from jax.experimental.pallas import tpu as pltpu
from jax.experimental.pallas import tpu_sc as plsc

VOCAB = 30000
EMB = 128
B = 4
S = 2048
EPS = 1e-12

NC = 2
NS = 16
NW = NC * NS
TOK = B * S
TPW = TOK // NW
IDXW = 128
NIDX = TPW // IDXW

TC_BLK = 2048


@functools.partial(
    pl.kernel,
    out_type=jax.ShapeDtypeStruct((TOK, EMB), jnp.float32),
    mesh=plsc.VectorSubcoreMesh(core_axis_name="c", subcore_axis_name="s"),
    scratch_types=[
        pltpu.VMEM((NIDX, IDXW), jnp.int32),
        pltpu.VMEM((TPW, EMB), jnp.float32),
        pltpu.SemaphoreType.DMA,
    ],
)
def _gather_add(ids_hbm, w_hbm, pos_hbm, out_hbm, idx_v, buf_v, gsem):
    cid = lax.axis_index("c")
    sid = lax.axis_index("s")
    wid = sid * NC + cid
    base = wid * TPW
    pos_base = lax.rem(base, S)

    pltpu.sync_copy(ids_hbm.at[pl.ds(wid * NIDX, NIDX)], idx_v)
    pltpu.sync_copy(pos_hbm.at[pl.ds(pos_base, TPW)], buf_v)

    cps = [
        pltpu.async_copy(w_hbm.at[idx_v.at[j]],
                         buf_v.at[pl.ds(j * IDXW, IDXW)], gsem, add=True)
        for j in range(NIDX)
    ]
    for cp in cps:
        cp.wait()

    pltpu.sync_copy(buf_v, out_hbm.at[pl.ds(base, TPW)])


def _ln_body(mid_ref, tte_ref, g_ref, b_ref, o_ref):
    x = mid_ref[...] + tte_ref[...]
    m = jnp.mean(x, axis=-1, keepdims=True)
    d = x - m
    var = jnp.mean(d * d, axis=-1, keepdims=True)
    o_ref[...] = d * lax.rsqrt(var + EPS) * g_ref[...] + b_ref[...]


_ln_call = pl.pallas_call(
    _ln_body,
    out_shape=jax.ShapeDtypeStruct((TOK, EMB), jnp.float32),
    grid=(TOK // TC_BLK,),
    in_specs=[
        pl.BlockSpec((TC_BLK, EMB), lambda i: (i, 0)),
        pl.BlockSpec((1, EMB), lambda i: (0, 0)),
        pl.BlockSpec((1, EMB), lambda i: (0, 0)),
        pl.BlockSpec((1, EMB), lambda i: (0, 0)),
    ],
    out_specs=pl.BlockSpec((TC_BLK, EMB), lambda i: (i, 0)),
)


def kernel(input_ids, weight, token_type_embeddings, position_embeddings,
           ln_gamma, ln_beta):
    ids = input_ids.astype(jnp.int32).reshape(TOK // IDXW, IDXW)
    mid = _gather_add(ids, weight, position_embeddings)
    out = _ln_call(mid,
                   token_type_embeddings[0].reshape(1, EMB),
                   ln_gamma.reshape(1, EMB),
                   ln_beta.reshape(1, EMB))
    return out.reshape(B, S, EMB)


# speedup vs baseline: 1.2766x; 1.0730x over previous
import functools

import jax
import jax.numpy as jnp
from jax import lax
from jax.experimental import pallas as pl
from jax.experimental.pallas import tpu as pltpu
from jax.experimental.pallas import tpu_sc as plsc

VOCAB = 30000
EMB = 128
B = 4
S = 2048
EPS = 1e-12

NC = 2
NS = 16
NW = NC * NS
TOK = B * S
TPW = TOK // NW
IDXW = 128
NIDX = TPW // IDXW


@functools.partial(
    pl.kernel,
    out_type=jax.ShapeDtypeStruct((TOK, EMB), jnp.float32),
    mesh=plsc.VectorSubcoreMesh(core_axis_name="c", subcore_axis_name="s"),
    scratch_types=[
        pltpu.VMEM((NIDX, IDXW), jnp.int32),
        pltpu.VMEM((TPW, EMB), jnp.float32),
        pltpu.SemaphoreType.DMA,
        pltpu.SemaphoreType.DMA,
        pltpu.SemaphoreType.DMA,
    ],
)
def _gather(ids_hbm, w_hbm, out_hbm, idx_v, rows_v, gsem0, gsem1, wsem):
    cid = lax.axis_index("c")
    sid = lax.axis_index("s")
    wid = sid * NC + cid
    base = wid * TPW

    pltpu.sync_copy(ids_hbm.at[pl.ds(wid * NIDX, NIDX)], idx_v)

    gsems = [gsem0, gsem1]
    gcps = [
        pltpu.async_copy(w_hbm.at[idx_v.at[j]],
                         rows_v.at[pl.ds(j * IDXW, IDXW)], gsems[j])
        for j in range(NIDX)
    ]
    wcps = []
    for j in range(NIDX):
        gcps[j].wait()
        wcps.append(pltpu.async_copy(
            rows_v.at[pl.ds(j * IDXW, IDXW)],
            out_hbm.at[pl.ds(base + j * IDXW, IDXW)], wsem))
    for cp in wcps:
        cp.wait()


def _ln_body(mid_ref, pos_ref, tte_ref, g_ref, b_ref, o_ref):
    x = mid_ref[...] + pos_ref[...] + tte_ref[...]
    m = jnp.mean(x, axis=-1, keepdims=True)
    d = x - m
    var = jnp.mean(d * d, axis=-1, keepdims=True)
    o_ref[...] = d * lax.rsqrt(var + EPS) * g_ref[...] + b_ref[...]


_ln_call = pl.pallas_call(
    _ln_body,
    out_shape=jax.ShapeDtypeStruct((TOK, EMB), jnp.float32),
    grid=(B,),
    in_specs=[
        pl.BlockSpec((S, EMB), lambda i: (i, 0)),
        pl.BlockSpec((S, EMB), lambda i: (0, 0)),
        pl.BlockSpec((1, EMB), lambda i: (0, 0)),
        pl.BlockSpec((1, EMB), lambda i: (0, 0)),
        pl.BlockSpec((1, EMB), lambda i: (0, 0)),
    ],
    out_specs=pl.BlockSpec((S, EMB), lambda i: (i, 0)),
)


def kernel(input_ids, weight, token_type_embeddings, position_embeddings,
           ln_gamma, ln_beta):
    ids = input_ids.astype(jnp.int32).reshape(TOK // IDXW, IDXW)
    mid = _gather(ids, weight)
    out = _ln_call(mid,
                   position_embeddings,
                   token_type_embeddings[0].reshape(1, EMB),
                   ln_gamma.reshape(1, EMB),
                   ln_beta.reshape(1, EMB))
    return out.reshape(B, S, EMB)
